# baseline (speedup 1.0000x reference)
import jax
import jax.numpy as jnp
from jax.experimental import pallas as pl

_STEPS = 32


def _discretize_block(x_ref, o_ref):
    x = x_ref[...]
    idx = jnp.floor(x * float(_STEPS)).astype(jnp.int32)
    c = jax.lax.broadcasted_iota(jnp.int32, o_ref.shape, 1)
    o_ref[...] = jnp.where(idx[:, None, :] == c, 1.0, 0.0).astype(jnp.float32)


@jax.jit
def kernel(x):
    B, P = x.shape
    Bb = 256
    return pl.pallas_call(
        _discretize_block,
        grid=(B // Bb,),
        in_specs=[pl.BlockSpec((Bb, P), lambda i: (i, 0))],
        out_specs=pl.BlockSpec((Bb, _STEPS, P), lambda i: (i, 0, 0)),
        out_shape=jax.ShapeDtypeStruct((B, _STEPS, P), jnp.float32),
    )(x)


# speedup vs baseline: 1.0835x; 1.0835x over previous
import jax
import jax.numpy as jnp
from jax.experimental import pallas as pl

_STEPS = 32


def _discretize_block(x_ref, o_ref):
    x = x_ref[...]
    idx = jnp.floor(x * float(_STEPS)).astype(jnp.int32)
    c = jax.lax.broadcasted_iota(jnp.int32, o_ref.shape, 1)
    o_ref[...] = jnp.where(idx[:, None, :] == c, 1.0, 0.0).astype(jnp.float32)


@jax.jit
def kernel(x):
    B, P = x.shape
    Bb = 512
    return pl.pallas_call(
        _discretize_block,
        grid=(B // Bb,),
        in_specs=[pl.BlockSpec((Bb, P), lambda i: (i, 0))],
        out_specs=pl.BlockSpec((Bb, _STEPS, P), lambda i: (i, 0, 0)),
        out_shape=jax.ShapeDtypeStruct((B, _STEPS, P), jnp.float32),
    )(x)
